# baseline (speedup 1.0000x reference)
import functools
import jax
import jax.numpy as jnp
from jax import lax
from jax.experimental import pallas as pl
from jax.experimental.pallas import tpu as pltpu
from jax.experimental.pallas import tpu_sc as plsc


@functools.lru_cache(maxsize=None)
def _make_gather(B, H, D, NC, NS):
    NW = NC * NS
    b_per_w = B // NW
    n_per_w = b_per_w * H
    chunk = b_per_w
    n_chunks = H
    TRD = D // 8
    TCB = b_per_w // 128
    mesh = plsc.VectorSubcoreMesh(core_axis_name="c", subcore_axis_name="s")

    @functools.partial(
        pl.kernel,
        mesh=mesh,
        out_type=jax.ShapeDtypeStruct((H, TRD, B // 128, 8, 128), jnp.float32),
        scratch_types=[
            pltpu.VMEM((b_per_w, H), jnp.int32),
            pltpu.VMEM((n_per_w,), jnp.int32),
            pltpu.VMEM((2, chunk, D), jnp.float32),
            pltpu.VMEM((2, TCB, 8, 128), jnp.float32),
            pltpu.SemaphoreType.DMA,
            pltpu.SemaphoreType.DMA,
            pltpu.SemaphoreType.DMA,
            pltpu.SemaphoreType.DMA,
        ],
        compiler_params=pltpu.CompilerParams(
            use_tc_tiling_on_sc=False, needs_layout_passes=False),
    )
    def k(x_hbm, table_hbm, out_hbm, xv, idx_t, rows_v, tbuf, sg0, sg1, st0, st1):
        wid = lax.axis_index("s") * NC + lax.axis_index("c")
        b_base = wid * b_per_w
        sem_g = [sg0, sg1]
        sem_t = [st0, st1]

        pltpu.sync_copy(x_hbm.at[pl.ds(b_base, b_per_w)], xv)

        def compact(g, carry):
            v = lax.iota(jnp.int32, 16) + g * 16
            t = plsc.load_gather(xv, [v % b_per_w, v // b_per_w])
            idx_t[pl.ds(g * 16, 16)] = t
            return carry

        lax.fori_loop(0, n_per_w // 16, compact, 0)

        def gather_desc(c, pb):
            return pltpu.make_async_copy(
                table_hbm.at[idx_t.at[pl.ds(c * chunk, chunk)]],
                rows_v.at[pb], sem_g[pb])

        def store_desc(h, tr, tb):
            return pltpu.make_async_copy(
                tbuf.at[tb],
                out_hbm.at[h, tr].at[pl.ds(wid * TCB, TCB)], sem_t[tb])

        def drain_t(tb):
            pltpu.make_async_copy(
                out_hbm.at[0, 0].at[pl.ds(0, TCB)], tbuf.at[tb],
                sem_t[tb]).wait()

        def transpose_chunk(h, pb, guard):
            for tr in range(TRD):
                tb = tr % 2
                if guard is None or tr >= 2:
                    drain_t(tb)
                else:
                    @pl.when(guard)
                    def _():
                        drain_t(tb)

                def one_tc(tc, carry):
                    rbase = tc * 128
                    for s in range(8):
                        cvec = jnp.full((16,), tr * 8 + s, jnp.int32)
                        for l0 in range(0, 128, 16):
                            rvec = lax.iota(jnp.int32, 16) + (rbase + l0)
                            t = plsc.load_gather(rows_v.at[pb], [rvec, cvec])
                            tbuf[tb, tc, s, pl.ds(l0, 16)] = t
                    return carry

                lax.fori_loop(0, TCB, one_tc, 0)
                store_desc(h, tr, tb).start()

        gather_desc(0, 0).start()

        def pair(i, carry):
            c0 = i * 2
            c1 = c0 + 1
            gather_desc(c0, 0).wait()
            gather_desc(c1, 1).start()
            transpose_chunk(c0, 0, i > 0)
            gather_desc(c1, 1).wait()

            @pl.when(i < n_chunks // 2 - 1)
            def _():
                gather_desc(c0 + 2, 0).start()

            transpose_chunk(c1, 1, None)
            return carry

        lax.fori_loop(0, n_chunks // 2, pair, 0)
        drain_t(0)
        drain_t(1)

    return k


def kernel(X, table):
    B, H = X.shape
    V, D = table.shape
    info = plsc.get_sparse_core_info()
    out5 = _make_gather(B, H, D, info.num_cores, info.num_subcores)(X, table)
    return jnp.transpose(out5, (2, 4, 0, 1, 3)).reshape(B, H, D)


# speedup vs baseline: 1.1403x; 1.1403x over previous
import functools
import jax
import jax.numpy as jnp
from jax import lax
from jax.experimental import pallas as pl
from jax.experimental.pallas import tpu as pltpu
from jax.experimental.pallas import tpu_sc as plsc


@functools.lru_cache(maxsize=None)
def _make_gather(B, H, D, NC, NS):
    NW = NC * NS
    b_per_w = B // NW
    n_per_w = b_per_w * H
    chunk = b_per_w
    n_chunks = H
    TRD = D // 8
    TCB = b_per_w // 128
    mesh = plsc.VectorSubcoreMesh(core_axis_name="c", subcore_axis_name="s")

    @functools.partial(
        pl.kernel,
        mesh=mesh,
        out_type=jax.ShapeDtypeStruct((H, TRD, B // 128, 8, 128), jnp.float32),
        scratch_types=[
            pltpu.VMEM((b_per_w, H), jnp.int32),
            pltpu.VMEM((n_per_w,), jnp.int32),
            pltpu.VMEM((2, chunk, D), jnp.float32),
            pltpu.VMEM((2, TRD, TCB, 8, 128), jnp.float32),
            pltpu.SemaphoreType.DMA,
            pltpu.SemaphoreType.DMA,
            pltpu.SemaphoreType.DMA,
            pltpu.SemaphoreType.DMA,
        ],
        compiler_params=pltpu.CompilerParams(
            use_tc_tiling_on_sc=False, needs_layout_passes=False,
            disable_bounds_checks=True),
    )
    def k(x_hbm, table_hbm, out_hbm, xv, idx_t, rows_v, tbuf, sg0, sg1, st0, st1):
        wid = lax.axis_index("s") * NC + lax.axis_index("c")
        b_base = wid * b_per_w
        sem_g = [sg0, sg1]
        sem_t = [st0, st1]

        pltpu.sync_copy(x_hbm.at[pl.ds(b_base, b_per_w)], xv)

        def compact(g, carry):
            v = lax.iota(jnp.int32, 16) + g * 16
            t = plsc.load_gather(xv, [v % b_per_w, v // b_per_w])
            idx_t[pl.ds(g * 16, 16)] = t
            return carry

        lax.fori_loop(0, n_per_w // 16, compact, 0)

        def gather_desc(c, pb):
            return pltpu.make_async_copy(
                table_hbm.at[idx_t.at[pl.ds(c * chunk, chunk)]],
                rows_v.at[pb], sem_g[pb])

        def store_desc(h, tb):
            return pltpu.make_async_copy(
                tbuf.at[tb],
                out_hbm.at[h].at[:, pl.ds(wid * TCB, TCB)], sem_t[tb])

        def drain_t(tb):
            pltpu.make_async_copy(
                out_hbm.at[0].at[:, pl.ds(0, TCB)], tbuf.at[tb],
                sem_t[tb]).wait()

        trv0 = lax.iota(jnp.int32, 16) // 8
        sv = lax.iota(jnp.int32, 16) % 8
        trv1 = trv0 + 2

        def transpose_chunk(h, pb, guard):
            if guard is None:
                drain_t(pb)
            else:
                @pl.when(guard)
                def _():
                    drain_t(pb)

            def one_j(j0, carry):
                for dj in range(4):
                    j = j0 * 4 + dj
                    lv = jnp.full((16,), 0, jnp.int32) + j
                    for tc in range(TCB):
                        tcv = jnp.full((16,), tc, jnp.int32)
                        r = tc * 128 + j
                        a = rows_v[pb, r, pl.ds(0, 16)]
                        bvec = rows_v[pb, r, pl.ds(16, 16)]
                        plsc.store_scatter(tbuf.at[pb], [trv0, tcv, sv, lv], a)
                        plsc.store_scatter(tbuf.at[pb], [trv1, tcv, sv, lv], bvec)
                return carry

            lax.fori_loop(0, 128 // 4, one_j, 0)
            store_desc(h, pb).start()

        gather_desc(0, 0).start()

        def pair(i, carry):
            c0 = i * 2
            c1 = c0 + 1
            gather_desc(c0, 0).wait()
            gather_desc(c1, 1).start()
            transpose_chunk(c0, 0, i > 0)
            gather_desc(c1, 1).wait()

            @pl.when(i < n_chunks // 2 - 1)
            def _():
                gather_desc(c0 + 2, 0).start()

            transpose_chunk(c1, 1, i > 0)
            return carry

        lax.fori_loop(0, n_chunks // 2, pair, 0)
        drain_t(0)
        drain_t(1)

    return k


def kernel(X, table):
    B, H = X.shape
    V, D = table.shape
    info = plsc.get_sparse_core_info()
    out5 = _make_gather(B, H, D, info.num_cores, info.num_subcores)(X, table)
    return jnp.transpose(out5, (2, 4, 0, 1, 3)).reshape(B, H, D)


# speedup vs baseline: 1.1557x; 1.0135x over previous
import functools
import jax
import jax.numpy as jnp
from jax import lax
from jax.experimental import pallas as pl
from jax.experimental.pallas import tpu as pltpu
from jax.experimental.pallas import tpu_sc as plsc


@functools.lru_cache(maxsize=None)
def _make_gather(B, H, D, NC, NS):
    NW = NC * NS
    b_per_w = B // NW
    chunk = b_per_w
    n_chunks = H
    TRD = D // 8
    TCB = b_per_w // 128
    mesh = plsc.VectorSubcoreMesh(core_axis_name="c", subcore_axis_name="s")

    @functools.partial(
        pl.kernel,
        mesh=mesh,
        out_type=jax.ShapeDtypeStruct((H, TRD, B // 128, 8, 128), jnp.float32),
        scratch_types=[
            pltpu.VMEM((H, chunk), jnp.int32),
            pltpu.VMEM((2, chunk, D), jnp.float32),
            pltpu.VMEM((2, TRD, TCB, 8, 128), jnp.float32),
            pltpu.SemaphoreType.DMA,
            pltpu.SemaphoreType.DMA,
            pltpu.SemaphoreType.DMA,
            pltpu.SemaphoreType.DMA,
        ],
        compiler_params=pltpu.CompilerParams(
            use_tc_tiling_on_sc=False, needs_layout_passes=False,
            disable_bounds_checks=True),
    )
    def k(xh_hbm, table_hbm, out_hbm, xv, rows_v, tbuf, sg0, sg1, st0, st1):
        wid = lax.axis_index("s") * NC + lax.axis_index("c")
        b_base = wid * b_per_w
        sem_g = [sg0, sg1]
        sem_t = [st0, st1]

        def idx_desc(h):
            return pltpu.make_async_copy(
                xh_hbm.at[pl.ds(h * B + b_base, chunk)], xv.at[h], sg0)

        for h in range(H):
            idx_desc(h).start()
        for h in range(H):
            idx_desc(h).wait()

        def gather_desc(c, pb):
            return pltpu.make_async_copy(
                table_hbm.at[xv.at[c]], rows_v.at[pb], sem_g[pb])

        def store_desc(h, tb):
            return pltpu.make_async_copy(
                tbuf.at[tb],
                out_hbm.at[h].at[:, pl.ds(wid * TCB, TCB)], sem_t[tb])

        def drain_t(tb):
            pltpu.make_async_copy(
                out_hbm.at[0].at[:, pl.ds(0, TCB)], tbuf.at[tb],
                sem_t[tb]).wait()

        trv0 = lax.iota(jnp.int32, 16) // 8
        sv = lax.iota(jnp.int32, 16) % 8
        trv1 = trv0 + 2

        def transpose_chunk(h, pb, guard):
            if guard is None:
                drain_t(pb)
            else:
                @pl.when(guard)
                def _():
                    drain_t(pb)

            def one_j(j0, carry):
                for dj in range(4):
                    j = j0 * 4 + dj
                    lv = jnp.full((16,), 0, jnp.int32) + j
                    for tc in range(TCB):
                        tcv = jnp.full((16,), tc, jnp.int32)
                        r = tc * 128 + j
                        a = rows_v[pb, r, pl.ds(0, 16)]
                        bvec = rows_v[pb, r, pl.ds(16, 16)]
                        plsc.store_scatter(tbuf.at[pb], [trv0, tcv, sv, lv], a)
                        plsc.store_scatter(tbuf.at[pb], [trv1, tcv, sv, lv], bvec)
                return carry

            lax.fori_loop(0, 128 // 4, one_j, 0)
            store_desc(h, pb).start()

        gather_desc(0, 0).start()

        def pair(i, carry):
            c0 = i * 2
            c1 = c0 + 1
            gather_desc(c0, 0).wait()
            gather_desc(c1, 1).start()
            transpose_chunk(c0, 0, i > 0)
            gather_desc(c1, 1).wait()

            @pl.when(i < n_chunks // 2 - 1)
            def _():
                gather_desc(c0 + 2, 0).start()

            transpose_chunk(c1, 1, i > 0)
            return carry

        lax.fori_loop(0, n_chunks // 2, pair, 0)
        drain_t(0)
        drain_t(1)

    return k


def kernel(X, table):
    B, H = X.shape
    V, D = table.shape
    info = plsc.get_sparse_core_info()
    xh = X.T.reshape(-1)
    out5 = _make_gather(B, H, D, info.num_cores, info.num_subcores)(xh, table)
    return jnp.transpose(out5, (2, 4, 0, 1, 3)).reshape(B, H, D)
